# baseline (speedup 1.0000x reference)
import jax
import jax.numpy as jnp
from jax.experimental import pallas as pl

ROWS, D = 8192, 768
BLOCK_ROWS = 4096


def _copy_body(w_ref, o_ref):
    o_ref[...] = w_ref[...]


def kernel(x, W):
    del x
    return pl.pallas_call(
        _copy_body,
        grid=(ROWS // BLOCK_ROWS,),
        in_specs=[pl.BlockSpec((BLOCK_ROWS, D), lambda i: (i, 0))],
        out_specs=pl.BlockSpec((BLOCK_ROWS, D), lambda i: (i, 0)),
        out_shape=jax.ShapeDtypeStruct((ROWS, D), jnp.float32),
    )(W)
